# baseline (speedup 1.0000x reference)
import jax
import jax.numpy as jnp
from jax import lax
from jax.experimental import pallas as pl
from jax.experimental.pallas import tpu as pltpu
from jax.experimental.pallas import tpu_sc as plsc

_B = 16384
_D = 64
_NC = 2
_NS = 16
_NW = _NC * _NS
_BPW = _B // _NW
_CHUNK = 128
_NCHUNK = _BPW // _CHUNK
_L = 16
_GROUPS = _BPW // _L


def _mf_body(user_hbm, item_hbm, uid_hbm, iid_hbm, out_hbm,
             uidx_v, iidx_v, urows_v, irows_v, part_v, out_v, sem_u, sem_v):
    wid = lax.axis_index("s") * _NC + lax.axis_index("c")
    base = wid * _BPW

    for c in range(_NCHUNK):
        pltpu.sync_copy(uid_hbm.at[pl.ds(base + c * _CHUNK, _CHUNK)], uidx_v.at[c])
        pltpu.sync_copy(iid_hbm.at[pl.ds(base + c * _CHUNK, _CHUNK)], iidx_v.at[c])

    copies = []
    for c in range(_NCHUNK):
        copies.append(pltpu.async_copy(
            user_hbm.at[uidx_v.at[c]], urows_v.at[pl.ds(c * _CHUNK, _CHUNK)], sem_u))
        copies.append(pltpu.async_copy(
            item_hbm.at[iidx_v.at[c]], irows_v.at[pl.ds(c * _CHUNK, _CHUNK)], sem_v))
    for cp in copies:
        cp.wait()

    lanes = lax.iota(jnp.int32, 16)

    def group_body(g, carry):
        row0 = g * _L
        for r in range(_L):
            row = row0 + r
            acc = urows_v[row, pl.ds(0, 16)] * irows_v[row, pl.ds(0, 16)]
            for c in range(1, _D // 16):
                acc = acc + (urows_v[row, pl.ds(c * 16, 16)]
                             * irows_v[row, pl.ds(c * 16, 16)])
            part_v[r] = acc
        s = jnp.zeros((16,), jnp.float32)
        for col in range(_L):
            s = s + plsc.load_gather(
                part_v, [lanes, jnp.full((16,), col, jnp.int32)])
        out_v[pl.ds(row0, _L)] = s
        return carry

    lax.fori_loop(0, _GROUPS, group_body, 0)
    pltpu.sync_copy(out_v, out_hbm.at[pl.ds(base, _BPW)])


def kernel(user_table, item_table, user_ids, item_ids):
    mesh = plsc.VectorSubcoreMesh(core_axis_name="c", subcore_axis_name="s")
    k = pl.kernel(
        _mf_body,
        mesh=mesh,
        out_type=jax.ShapeDtypeStruct((_B,), jnp.float32),
        scratch_types=[
            pltpu.VMEM((_NCHUNK, _CHUNK), jnp.int32),
            pltpu.VMEM((_NCHUNK, _CHUNK), jnp.int32),
            pltpu.VMEM((_BPW, _D), jnp.float32),
            pltpu.VMEM((_BPW, _D), jnp.float32),
            pltpu.VMEM((_L, _L), jnp.float32),
            pltpu.VMEM((_BPW,), jnp.float32),
            pltpu.SemaphoreType.DMA,
            pltpu.SemaphoreType.DMA,
        ],
        compiler_params=pltpu.CompilerParams(
            needs_layout_passes=False, use_tc_tiling_on_sc=False),
    )
    return k(user_table, item_table,
             user_ids.astype(jnp.int32), item_ids.astype(jnp.int32))
